# Initial kernel scaffold; baseline (speedup 1.0000x reference)
import jax
import jax.numpy as jnp
from jax.experimental import pallas as pl


def kernel(points_2d, camera_indices, grouping_indices, point_indices, camera_pps, rel_poses, intrs, points_3d, ref_poses):
    raise NotImplementedError("write your pallas kernel here")


# speedup vs baseline: 2.6957x; 2.6957x over previous
import functools

import jax
import jax.numpy as jnp
from jax import lax
from jax.experimental import pallas as pl
from jax.experimental.pallas import tpu as pltpu
from jax.experimental.pallas import tpu_sc as plsc

CHUNK = 128
_L = 16


def _i32(x):
    return jnp.full((_L,), x, dtype=jnp.int32)


def _qmul(lq, rq):
    lx, ly, lz, lw = lq
    rx, ry, rz, rw = rq
    w = lw * rw - lx * rx - ly * ry - lz * rz
    x = lw * rx + lx * rw + ly * rz - lz * ry
    y = lw * ry - lx * rz + ly * rw + lz * rx
    z = lw * rz + lx * ry - ly * rx + lz * rw
    return (x, y, z, w)


def _qrot(q, v):
    qx, qy, qz, qw = q
    vx, vy, vz = v
    tx = 2.0 * (qy * vz - qz * vy)
    ty = 2.0 * (qz * vx - qx * vz)
    tz = 2.0 * (qx * vy - qy * vx)
    ox = vx + qw * tx + (qy * tz - qz * ty)
    oy = vy + qw * ty + (qz * tx - qx * tz)
    oz = vz + qw * tz + (qx * ty - qy * tx)
    return (ox, oy, oz)


def _make_sc_call(n, num_chunks):
    nc, ns = 2, 16
    nw = nc * ns
    mesh = plsc.VectorSubcoreMesh(core_axis_name="c", subcore_axis_name="s",
                                  num_cores=nc, num_subcores=ns)

    @functools.partial(
        pl.kernel,
        mesh=mesh,
        compiler_params=pltpu.CompilerParams(
            needs_layout_passes=False,
            use_tc_tiling_on_sc=False,
        ),
        out_type=jax.ShapeDtypeStruct((n, 2), jnp.float32),
        scratch_types=[
            pltpu.VMEM((8, 8), jnp.float32),
            pltpu.VMEM((8, 4), jnp.float32),
            pltpu.VMEM((CHUNK, 2), jnp.int32),
            pltpu.VMEM((CHUNK,), jnp.int32),
            pltpu.VMEM((CHUNK,), jnp.int32),
            pltpu.VMEM((CHUNK,), jnp.int32),
            pltpu.VMEM((CHUNK, 8), jnp.float32),
            pltpu.VMEM((CHUNK, 8), jnp.float32),
            pltpu.VMEM((CHUNK, 2), jnp.float32),
            pltpu.VMEM((CHUNK, 2), jnp.float32),
            pltpu.SemaphoreType.DMA,
            pltpu.SemaphoreType.DMA,
        ],
    )
    def sc_call(ref8, pt8, rel8, cam4, grouping, cidx, pidx, p2d, out,
                rel_v, cam_v, grp_v, gidx_v, pidx_v, cidx_v,
                ref_v, pt_v, p2d_v, out_v, sem0, sem1):
        wid = lax.axis_index("s") * nc + lax.axis_index("c")
        pltpu.sync_copy(rel8, rel_v)
        pltpu.sync_copy(cam4, cam_v)
        iota = lax.iota(jnp.int32, _L)

        @pl.loop(wid, num_chunks, step=nw)
        def _chunk(c):
            base = c * CHUNK
            pltpu.sync_copy(grouping.at[pl.ds(base, CHUNK)], grp_v)
            pltpu.sync_copy(pidx.at[pl.ds(base, CHUNK)], pidx_v)
            pltpu.sync_copy(cidx.at[pl.ds(base, CHUNK)], cidx_v)
            pltpu.sync_copy(p2d.at[pl.ds(base, CHUNK)], p2d_v)
            for j in range(CHUNK // _L):
                row = iota + (j * _L)
                g = plsc.load_gather(grp_v, [row, _i32(0)])
                gidx_v[pl.ds(j * _L, _L)] = g
            ref_cp = pltpu.async_copy(ref8.at[gidx_v], ref_v, sem0)
            pt_cp = pltpu.async_copy(pt8.at[pidx_v], pt_v, sem1)
            ref_cp.wait()
            pt_cp.wait()
            for j in range(CHUNK // _L):
                row = iota + (j * _L)
                m = plsc.load_gather(grp_v, [row, _i32(1)])
                cam = cidx_v[pl.ds(j * _L, _L)]
                rt = tuple(plsc.load_gather(ref_v, [row, _i32(k)]) for k in range(3))
                rq = tuple(plsc.load_gather(ref_v, [row, _i32(k)]) for k in range(3, 7))
                lt = tuple(plsc.load_gather(rel_v, [m, _i32(k)]) for k in range(3))
                lq = tuple(plsc.load_gather(rel_v, [m, _i32(k)]) for k in range(3, 7))
                p = tuple(plsc.load_gather(pt_v, [row, _i32(k)]) for k in range(3))
                q_img = _qmul(lq, rq)
                rot = _qrot(lq, rt)
                t_img = (lt[0] + rot[0], lt[1] + rot[1], lt[2] + rot[2])
                pc = _qrot(q_img, p)
                pcx = pc[0] + t_img[0]
                pcy = pc[1] + t_img[1]
                pcz = pc[2] + t_img[2]
                fx = plsc.load_gather(cam_v, [cam, _i32(0)])
                fy = plsc.load_gather(cam_v, [cam, _i32(1)])
                ppx = plsc.load_gather(cam_v, [cam, _i32(2)])
                ppy = plsc.load_gather(cam_v, [cam, _i32(3)])
                ox = plsc.load_gather(p2d_v, [row, _i32(0)])
                oy = plsc.load_gather(p2d_v, [row, _i32(1)])
                rx = fx * pcx / pcz + ppx - ox
                ry = fy * pcy / pcz + ppy - oy
                plsc.store_scatter(out_v, [row, _i32(0)], rx)
                plsc.store_scatter(out_v, [row, _i32(1)], ry)
            pltpu.sync_copy(out_v, out.at[pl.ds(base, CHUNK)])

    return sc_call


@jax.jit
def _run(points_2d, camera_indices, grouping_indices, point_indices,
         camera_pps, rel_poses, intrs, points_3d, ref_poses):
    n = points_2d.shape[0]
    num_chunks = n // CHUNK
    g = ref_poses.shape[0]
    p = points_3d.shape[0]
    ref8 = jnp.concatenate(
        [ref_poses, jnp.zeros((g, 1), jnp.float32)], axis=1)
    pt8 = jnp.concatenate(
        [points_3d, jnp.zeros((p, 5), jnp.float32)], axis=1)
    rel8 = jnp.concatenate(
        [rel_poses, jnp.zeros((rel_poses.shape[0], 1), jnp.float32)], axis=1)
    cam4 = jnp.concatenate([intrs, camera_pps], axis=1)
    sc_call = _make_sc_call(n, num_chunks)
    return sc_call(ref8, pt8, rel8, cam4,
                   grouping_indices.astype(jnp.int32),
                   camera_indices.astype(jnp.int32),
                   point_indices.astype(jnp.int32),
                   points_2d)


def kernel(points_2d, camera_indices, grouping_indices, point_indices,
           camera_pps, rel_poses, intrs, points_3d, ref_poses):
    return _run(points_2d, camera_indices, grouping_indices, point_indices,
                camera_pps, rel_poses, intrs, points_3d, ref_poses)


# speedup vs baseline: 13.4099x; 4.9745x over previous
import functools

import jax
import jax.numpy as jnp
from jax import lax
from jax.experimental import pallas as pl
from jax.experimental.pallas import tpu as pltpu
from jax.experimental.pallas import tpu_sc as plsc

CHUNK = 128
_L = 16
_NC, _NS = 2, 16
_NW = _NC * _NS

_PTA = 160
_REFA = 80


def _i32(x):
    return jnp.full((_L,), x, dtype=jnp.int32)


def _qmul(lq, rq):
    lx, ly, lz, lw = lq
    rx, ry, rz, rw = rq
    w = lw * rw - lx * rx - ly * ry - lz * rz
    x = lw * rx + lx * rw + ly * rz - lz * ry
    y = lw * ry - lx * rz + ly * rw + lz * rx
    z = lw * rz + lx * ry - ly * rx + lz * rw
    return (x, y, z, w)


def _qrot(q, v):
    qx, qy, qz, qw = q
    vx, vy, vz = v
    tx = 2.0 * (qy * vz - qz * vy)
    ty = 2.0 * (qz * vx - qx * vz)
    tz = 2.0 * (qx * vy - qy * vx)
    ox = vx + qw * tx + (qy * tz - qz * ty)
    oy = vy + qw * ty + (qz * tx - qx * tz)
    oz = vz + qw * tz + (qx * ty - qy * tx)
    return (ox, oy, oz)


def _mesh():
    return plsc.VectorSubcoreMesh(core_axis_name="c", subcore_axis_name="s",
                                  num_cores=_NC, num_subcores=_NS)


def _make_pack_tables(npts, ngroups):
    npt_chunks = npts // _PTA
    nref_chunks = ngroups // _REFA

    @functools.partial(
        pl.kernel,
        mesh=_mesh(),
        compiler_params=pltpu.CompilerParams(
            needs_layout_passes=False,
            use_tc_tiling_on_sc=False,
        ),
        out_type=[jax.ShapeDtypeStruct((npts, 8), jnp.float32),
                  jax.ShapeDtypeStruct((ngroups, 8), jnp.float32)],
        scratch_types=[
            pltpu.VMEM((_PTA,), jnp.float32),
            pltpu.VMEM((_PTA,), jnp.float32),
            pltpu.VMEM((_PTA,), jnp.float32),
            pltpu.VMEM((_PTA, 8), jnp.float32),
            [pltpu.VMEM((_REFA,), jnp.float32) for _ in range(7)],
            pltpu.VMEM((_REFA, 8), jnp.float32),
        ],
    )
    def pack(p3x, p3y, p3z, r0, r1, r2, r3, r4, r5, r6, pt8, ref8,
             xb, yb, zb, ptb, rcols, refb):
        wid = lax.axis_index("s") * _NC + lax.axis_index("c")
        iota = lax.iota(jnp.int32, _L)
        rsrcs = (r0, r1, r2, r3, r4, r5, r6)

        @pl.loop(wid, npt_chunks, step=_NW)
        def _pt(c):
            base = c * _PTA
            pltpu.sync_copy(p3x.at[pl.ds(base, _PTA)], xb)
            pltpu.sync_copy(p3y.at[pl.ds(base, _PTA)], yb)
            pltpu.sync_copy(p3z.at[pl.ds(base, _PTA)], zb)
            for j in range(_PTA // _L):
                row = iota + (j * _L)
                sl = pl.ds(j * _L, _L)
                plsc.store_scatter(ptb, [row, _i32(0)], xb[sl])
                plsc.store_scatter(ptb, [row, _i32(1)], yb[sl])
                plsc.store_scatter(ptb, [row, _i32(2)], zb[sl])
            pltpu.sync_copy(ptb, pt8.at[pl.ds(base, _PTA)])

        @pl.loop(wid, nref_chunks, step=_NW)
        def _ref(c):
            base = c * _REFA
            for k in range(7):
                pltpu.sync_copy(rsrcs[k].at[pl.ds(base, _REFA)], rcols[k])
            for j in range(_REFA // _L):
                row = iota + (j * _L)
                sl = pl.ds(j * _L, _L)
                for k in range(7):
                    plsc.store_scatter(refb, [row, _i32(k)], rcols[k][sl])
            pltpu.sync_copy(refb, ref8.at[pl.ds(base, _REFA)])

    return pack


def _make_main(n):
    num_chunks = n // CHUNK

    @functools.partial(
        pl.kernel,
        mesh=_mesh(),
        compiler_params=pltpu.CompilerParams(
            needs_layout_passes=False,
            use_tc_tiling_on_sc=False,
        ),
        out_type=jax.ShapeDtypeStruct((2 * n,), jnp.float32),
        scratch_types=[
            pltpu.VMEM((8, 8), jnp.float32),
            pltpu.VMEM((8, 4), jnp.float32),
            pltpu.VMEM((CHUNK,), jnp.int32),
            pltpu.VMEM((CHUNK,), jnp.int32),
            pltpu.VMEM((CHUNK,), jnp.int32),
            pltpu.VMEM((CHUNK,), jnp.int32),
            pltpu.VMEM((CHUNK, 8), jnp.float32),
            pltpu.VMEM((CHUNK, 8), jnp.float32),
            pltpu.VMEM((2 * CHUNK,), jnp.float32),
            pltpu.VMEM((2 * CHUNK,), jnp.float32),
            pltpu.SemaphoreType.DMA,
            pltpu.SemaphoreType.DMA,
        ],
    )
    def main(ref8, pt8, rel8, cam4, gm_flat, cidx, pidx, p2d_flat, out,
             rel_v, cam_v, gbuf, mbuf, pbuf, cbuf,
             ref_v, pt_v, p2buf, obuf, sem0, sem1):
        wid = lax.axis_index("s") * _NC + lax.axis_index("c")
        pltpu.sync_copy(rel8, rel_v)
        pltpu.sync_copy(cam4, cam_v)
        iota = lax.iota(jnp.int32, _L)

        @pl.loop(wid, num_chunks, step=_NW)
        def _chunk(k):
            base2 = k * (2 * CHUNK)
            base = k * CHUNK
            pltpu.sync_copy(gm_flat.at[pl.ds(base2, CHUNK)], gbuf)
            pltpu.sync_copy(gm_flat.at[pl.ds(base2 + CHUNK, CHUNK)], mbuf)
            pltpu.sync_copy(pidx.at[pl.ds(base, CHUNK)], pbuf)
            pltpu.sync_copy(cidx.at[pl.ds(base, CHUNK)], cbuf)
            pltpu.sync_copy(p2d_flat.at[pl.ds(base2, 2 * CHUNK)], p2buf)
            ref_cp = pltpu.async_copy(ref8.at[gbuf], ref_v, sem0)
            pt_cp = pltpu.async_copy(pt8.at[pbuf], pt_v, sem1)
            ref_cp.wait()
            pt_cp.wait()
            for j in range(CHUNK // _L):
                row = iota + (j * _L)
                sl = pl.ds(j * _L, _L)
                sl2 = pl.ds(CHUNK + j * _L, _L)
                m = mbuf[sl]
                cam = cbuf[sl]
                rt = tuple(plsc.load_gather(ref_v, [row, _i32(c)]) for c in range(3))
                rq = tuple(plsc.load_gather(ref_v, [row, _i32(c)]) for c in range(3, 7))
                lt = tuple(plsc.load_gather(rel_v, [m, _i32(c)]) for c in range(3))
                lq = tuple(plsc.load_gather(rel_v, [m, _i32(c)]) for c in range(3, 7))
                p = tuple(plsc.load_gather(pt_v, [row, _i32(c)]) for c in range(3))
                q_img = _qmul(lq, rq)
                rot = _qrot(lq, rt)
                t_img = (lt[0] + rot[0], lt[1] + rot[1], lt[2] + rot[2])
                pc = _qrot(q_img, p)
                pcx = pc[0] + t_img[0]
                pcy = pc[1] + t_img[1]
                pcz = pc[2] + t_img[2]
                fx = plsc.load_gather(cam_v, [cam, _i32(0)])
                fy = plsc.load_gather(cam_v, [cam, _i32(1)])
                ppx = plsc.load_gather(cam_v, [cam, _i32(2)])
                ppy = plsc.load_gather(cam_v, [cam, _i32(3)])
                obuf[sl] = fx * pcx / pcz + ppx - p2buf[sl]
                obuf[sl2] = fy * pcy / pcz + ppy - p2buf[sl2]
            pltpu.sync_copy(obuf, out.at[pl.ds(base2, 2 * CHUNK)])

    return main


@jax.jit
def _run(points_2d, camera_indices, grouping_indices, point_indices,
         camera_pps, rel_poses, intrs, points_3d, ref_poses):
    n = points_2d.shape[0]
    nb = n // CHUNK
    npts = points_3d.shape[0]
    ngroups = ref_poses.shape[0]
    pack = _make_pack_tables(npts, ngroups)
    pt8, ref8 = pack(points_3d[:, 0], points_3d[:, 1], points_3d[:, 2],
                     *(ref_poses[:, i] for i in range(7)))
    rel8 = jnp.concatenate(
        [rel_poses, jnp.zeros((rel_poses.shape[0], 1), jnp.float32)], axis=1)
    cam4 = jnp.concatenate([intrs, camera_pps], axis=1)
    gm_flat = (grouping_indices.astype(jnp.int32)
               .reshape(nb, CHUNK, 2).transpose(0, 2, 1).reshape(-1))
    p2d_flat = points_2d.reshape(nb, CHUNK, 2).transpose(0, 2, 1).reshape(-1)
    main = _make_main(n)
    out_flat = main(ref8, pt8, rel8, cam4, gm_flat,
                    camera_indices.astype(jnp.int32),
                    point_indices.astype(jnp.int32),
                    p2d_flat)
    return out_flat.reshape(nb, 2, CHUNK).transpose(0, 2, 1).reshape(n, 2)


def kernel(points_2d, camera_indices, grouping_indices, point_indices,
           camera_pps, rel_poses, intrs, points_3d, ref_poses):
    return _run(points_2d, camera_indices, grouping_indices, point_indices,
                camera_pps, rel_poses, intrs, points_3d, ref_poses)


# speedup vs baseline: 42.6194x; 3.1782x over previous
import functools

import jax
import jax.numpy as jnp
from jax import lax
from jax.experimental import pallas as pl
from jax.experimental.pallas import tpu as pltpu
from jax.experimental.pallas import tpu_sc as plsc

BLK = 128
CHUNK = 640
_SUB = CHUNK // BLK
_L = 16
_NC, _NS = 2, 16
_NW = _NC * _NS

_PTA = 800
_REFA = 400


def _i32(x):
    return jnp.full((_L,), x, dtype=jnp.int32)


def _qmul(lq, rq):
    lx, ly, lz, lw = lq
    rx, ry, rz, rw = rq
    w = lw * rw - lx * rx - ly * ry - lz * rz
    x = lw * rx + lx * rw + ly * rz - lz * ry
    y = lw * ry - lx * rz + ly * rw + lz * rx
    z = lw * rz + lx * ry - ly * rx + lz * rw
    return (x, y, z, w)


def _qrot(q, v):
    qx, qy, qz, qw = q
    vx, vy, vz = v
    tx = 2.0 * (qy * vz - qz * vy)
    ty = 2.0 * (qz * vx - qx * vz)
    tz = 2.0 * (qx * vy - qy * vx)
    ox = vx + qw * tx + (qy * tz - qz * ty)
    oy = vy + qw * ty + (qz * tx - qx * tz)
    oz = vz + qw * tz + (qx * ty - qy * tx)
    return (ox, oy, oz)


def _mesh():
    return plsc.VectorSubcoreMesh(core_axis_name="c", subcore_axis_name="s",
                                  num_cores=_NC, num_subcores=_NS)


def _make_pack_tables(npts, ngroups):
    npt_chunks = npts // _PTA
    nref_chunks = ngroups // _REFA

    @functools.partial(
        pl.kernel,
        mesh=_mesh(),
        compiler_params=pltpu.CompilerParams(
            needs_layout_passes=False,
            use_tc_tiling_on_sc=False,
        ),
        out_type=[jax.ShapeDtypeStruct((npts, 8), jnp.float32),
                  jax.ShapeDtypeStruct((ngroups, 8), jnp.float32)],
        scratch_types=[
            pltpu.VMEM((_PTA,), jnp.float32),
            pltpu.VMEM((_PTA,), jnp.float32),
            pltpu.VMEM((_PTA,), jnp.float32),
            pltpu.VMEM((_PTA, 8), jnp.float32),
            [pltpu.VMEM((_REFA,), jnp.float32) for _ in range(7)],
            pltpu.VMEM((_REFA, 8), jnp.float32),
            pltpu.SemaphoreType.DMA,
        ],
    )
    def pack(p3x, p3y, p3z, r0, r1, r2, r3, r4, r5, r6, pt8, ref8,
             xb, yb, zb, ptb, rcols, refb, sem):
        wid = lax.axis_index("s") * _NC + lax.axis_index("c")
        iota = lax.iota(jnp.int32, _L)
        rsrcs = (r0, r1, r2, r3, r4, r5, r6)

        @pl.loop(wid, npt_chunks, step=_NW)
        def _pt(c):
            base = c * _PTA
            cx = pltpu.async_copy(p3x.at[pl.ds(base, _PTA)], xb, sem)
            cy = pltpu.async_copy(p3y.at[pl.ds(base, _PTA)], yb, sem)
            cz = pltpu.async_copy(p3z.at[pl.ds(base, _PTA)], zb, sem)
            cx.wait()
            cy.wait()
            cz.wait()
            for j in range(_PTA // _L):
                row = iota + (j * _L)
                sl = pl.ds(j * _L, _L)
                plsc.store_scatter(ptb, [row, _i32(0)], xb[sl])
                plsc.store_scatter(ptb, [row, _i32(1)], yb[sl])
                plsc.store_scatter(ptb, [row, _i32(2)], zb[sl])
            pltpu.sync_copy(ptb, pt8.at[pl.ds(base, _PTA)])

        @pl.loop(wid, nref_chunks, step=_NW)
        def _ref(c):
            base = c * _REFA
            cps = [pltpu.async_copy(rsrcs[k].at[pl.ds(base, _REFA)],
                                    rcols[k], sem) for k in range(7)]
            for cp in cps:
                cp.wait()
            for j in range(_REFA // _L):
                row = iota + (j * _L)
                sl = pl.ds(j * _L, _L)
                for k in range(7):
                    plsc.store_scatter(refb, [row, _i32(k)], rcols[k][sl])
            pltpu.sync_copy(refb, ref8.at[pl.ds(base, _REFA)])

    return pack


def _make_main(n):
    num_chunks = n // CHUNK

    @functools.partial(
        pl.kernel,
        mesh=_mesh(),
        compiler_params=pltpu.CompilerParams(
            needs_layout_passes=False,
            use_tc_tiling_on_sc=False,
        ),
        out_type=jax.ShapeDtypeStruct((2 * n,), jnp.float32),
        scratch_types=[
            pltpu.VMEM((8, 8), jnp.float32),
            pltpu.VMEM((8, 4), jnp.float32),
            pltpu.VMEM((2 * CHUNK,), jnp.int32),
            pltpu.VMEM((CHUNK,), jnp.int32),
            pltpu.VMEM((CHUNK,), jnp.int32),
            pltpu.VMEM((CHUNK, 8), jnp.float32),
            pltpu.VMEM((CHUNK, 8), jnp.float32),
            pltpu.VMEM((2 * CHUNK,), jnp.float32),
            pltpu.VMEM((2 * CHUNK,), jnp.float32),
            pltpu.SemaphoreType.DMA,
            pltpu.SemaphoreType.DMA,
        ],
    )
    def main(ref8, pt8, rel8, cam4, gm_flat, cidx, pidx, p2d_flat, out,
             rel_v, cam_v, gmbuf, pbuf, cbuf,
             ref_v, pt_v, p2buf, obuf, sem_in, sem_tab):
        wid = lax.axis_index("s") * _NC + lax.axis_index("c")
        pltpu.sync_copy(rel8, rel_v)
        pltpu.sync_copy(cam4, cam_v)
        iota = lax.iota(jnp.int32, _L)

        @pl.loop(wid, num_chunks, step=_NW)
        def _chunk(k):
            base2 = k * (2 * CHUNK)
            base = k * CHUNK
            cgm = pltpu.async_copy(gm_flat.at[pl.ds(base2, 2 * CHUNK)],
                                   gmbuf, sem_in)
            cpi = pltpu.async_copy(pidx.at[pl.ds(base, CHUNK)], pbuf, sem_in)
            cci = pltpu.async_copy(cidx.at[pl.ds(base, CHUNK)], cbuf, sem_in)
            cp2 = pltpu.async_copy(p2d_flat.at[pl.ds(base2, 2 * CHUNK)],
                                   p2buf, sem_in)
            cgm.wait()
            cpi.wait()
            ref_cps = [pltpu.async_copy(
                ref8.at[gmbuf.at[pl.ds(i * 2 * BLK, BLK)]],
                ref_v.at[pl.ds(i * BLK, BLK)], sem_tab)
                for i in range(_SUB)]
            pt_cps = [pltpu.async_copy(
                pt8.at[pbuf.at[pl.ds(i * BLK, BLK)]],
                pt_v.at[pl.ds(i * BLK, BLK)], sem_tab)
                for i in range(_SUB)]
            cci.wait()
            cp2.wait()
            for cp in ref_cps:
                cp.wait()
            for cp in pt_cps:
                cp.wait()

            @pl.loop(0, CHUNK // _L)
            def _grp(jj):
                blk = jj // 8
                jw = jj - blk * 8
                row = iota + (jj * _L)
                sl = pl.ds(jj * _L, _L)
                slx = pl.ds(blk * 2 * BLK + jw * _L, _L)
                sly = pl.ds(blk * 2 * BLK + BLK + jw * _L, _L)
                m = gmbuf[sly]
                cam = cbuf[sl]
                rt = tuple(plsc.load_gather(ref_v, [row, _i32(c)]) for c in range(3))
                rq = tuple(plsc.load_gather(ref_v, [row, _i32(c)]) for c in range(3, 7))
                lt = tuple(plsc.load_gather(rel_v, [m, _i32(c)]) for c in range(3))
                lq = tuple(plsc.load_gather(rel_v, [m, _i32(c)]) for c in range(3, 7))
                p = tuple(plsc.load_gather(pt_v, [row, _i32(c)]) for c in range(3))
                q_img = _qmul(lq, rq)
                rot = _qrot(lq, rt)
                t_img = (lt[0] + rot[0], lt[1] + rot[1], lt[2] + rot[2])
                pc = _qrot(q_img, p)
                pcx = pc[0] + t_img[0]
                pcy = pc[1] + t_img[1]
                pcz = pc[2] + t_img[2]
                fx = plsc.load_gather(cam_v, [cam, _i32(0)])
                fy = plsc.load_gather(cam_v, [cam, _i32(1)])
                ppx = plsc.load_gather(cam_v, [cam, _i32(2)])
                ppy = plsc.load_gather(cam_v, [cam, _i32(3)])
                obuf[slx] = fx * pcx / pcz + ppx - p2buf[slx]
                obuf[sly] = fy * pcy / pcz + ppy - p2buf[sly]

            pltpu.sync_copy(obuf, out.at[pl.ds(base2, 2 * CHUNK)])

    return main


@jax.jit
def _run(points_2d, camera_indices, grouping_indices, point_indices,
         camera_pps, rel_poses, intrs, points_3d, ref_poses):
    n = points_2d.shape[0]
    nb = n // BLK
    npts = points_3d.shape[0]
    ngroups = ref_poses.shape[0]
    pack = _make_pack_tables(npts, ngroups)
    pt8, ref8 = pack(points_3d[:, 0], points_3d[:, 1], points_3d[:, 2],
                     *(ref_poses[:, i] for i in range(7)))
    rel8 = jnp.concatenate(
        [rel_poses, jnp.zeros((rel_poses.shape[0], 1), jnp.float32)], axis=1)
    cam4 = jnp.concatenate([intrs, camera_pps], axis=1)
    gm_flat = (grouping_indices.astype(jnp.int32)
               .reshape(nb, BLK, 2).transpose(0, 2, 1).reshape(-1))
    p2d_flat = points_2d.reshape(nb, BLK, 2).transpose(0, 2, 1).reshape(-1)
    main = _make_main(n)
    out_flat = main(ref8, pt8, rel8, cam4, gm_flat,
                    camera_indices.astype(jnp.int32),
                    point_indices.astype(jnp.int32),
                    p2d_flat)
    return out_flat.reshape(nb, 2, BLK).transpose(0, 2, 1).reshape(n, 2)


def kernel(points_2d, camera_indices, grouping_indices, point_indices,
           camera_pps, rel_poses, intrs, points_3d, ref_poses):
    return _run(points_2d, camera_indices, grouping_indices, point_indices,
                camera_pps, rel_poses, intrs, points_3d, ref_poses)


# speedup vs baseline: 57.7004x; 1.3539x over previous
import functools

import jax
import jax.numpy as jnp
from jax import lax
from jax.experimental import pallas as pl
from jax.experimental.pallas import tpu as pltpu
from jax.experimental.pallas import tpu_sc as plsc

BLK = 128
CHUNK = 640
_SUB = CHUNK // BLK
_L = 16
_NC, _NS = 2, 16
_NW = _NC * _NS

_PTA = 800
_REFA = 400


def _i32(x):
    return jnp.full((_L,), x, dtype=jnp.int32)


def _qmul(lq, rq):
    lx, ly, lz, lw = lq
    rx, ry, rz, rw = rq
    w = lw * rw - lx * rx - ly * ry - lz * rz
    x = lw * rx + lx * rw + ly * rz - lz * ry
    y = lw * ry - lx * rz + ly * rw + lz * rx
    z = lw * rz + lx * ry - ly * rx + lz * rw
    return (x, y, z, w)


def _qrot(q, v):
    qx, qy, qz, qw = q
    vx, vy, vz = v
    tx = 2.0 * (qy * vz - qz * vy)
    ty = 2.0 * (qz * vx - qx * vz)
    tz = 2.0 * (qx * vy - qy * vx)
    ox = vx + qw * tx + (qy * tz - qz * ty)
    oy = vy + qw * ty + (qz * tx - qx * tz)
    oz = vz + qw * tz + (qx * ty - qy * tx)
    return (ox, oy, oz)


def _mesh():
    return plsc.VectorSubcoreMesh(core_axis_name="c", subcore_axis_name="s",
                                  num_cores=_NC, num_subcores=_NS)


def _make_pack_tables(npts, ngroups):
    npt_chunks = npts // _PTA
    nref_chunks = ngroups // _REFA

    @functools.partial(
        pl.kernel,
        mesh=_mesh(),
        compiler_params=pltpu.CompilerParams(
            needs_layout_passes=False,
            use_tc_tiling_on_sc=False,
        ),
        out_type=[jax.ShapeDtypeStruct((npts, 8), jnp.float32),
                  jax.ShapeDtypeStruct((ngroups, 8), jnp.float32)],
        scratch_types=[
            pltpu.VMEM((_PTA,), jnp.float32),
            pltpu.VMEM((_PTA,), jnp.float32),
            pltpu.VMEM((_PTA,), jnp.float32),
            pltpu.VMEM((_PTA, 8), jnp.float32),
            [pltpu.VMEM((_REFA,), jnp.float32) for _ in range(7)],
            pltpu.VMEM((_REFA, 8), jnp.float32),
            pltpu.SemaphoreType.DMA,
        ],
    )
    def pack(p3x, p3y, p3z, r0, r1, r2, r3, r4, r5, r6, pt8, ref8,
             xb, yb, zb, ptb, rcols, refb, sem):
        wid = lax.axis_index("s") * _NC + lax.axis_index("c")
        iota = lax.iota(jnp.int32, _L)
        rsrcs = (r0, r1, r2, r3, r4, r5, r6)

        @pl.loop(wid, npt_chunks, step=_NW)
        def _pt(c):
            base = c * _PTA
            cx = pltpu.async_copy(p3x.at[pl.ds(base, _PTA)], xb, sem)
            cy = pltpu.async_copy(p3y.at[pl.ds(base, _PTA)], yb, sem)
            cz = pltpu.async_copy(p3z.at[pl.ds(base, _PTA)], zb, sem)
            cx.wait()
            cy.wait()
            cz.wait()
            for j in range(_PTA // _L):
                row = iota + (j * _L)
                sl = pl.ds(j * _L, _L)
                plsc.store_scatter(ptb, [row, _i32(0)], xb[sl])
                plsc.store_scatter(ptb, [row, _i32(1)], yb[sl])
                plsc.store_scatter(ptb, [row, _i32(2)], zb[sl])
            pltpu.sync_copy(ptb, pt8.at[pl.ds(base, _PTA)])

        @pl.loop(wid, nref_chunks, step=_NW)
        def _ref(c):
            base = c * _REFA
            cps = [pltpu.async_copy(rsrcs[k].at[pl.ds(base, _REFA)],
                                    rcols[k], sem) for k in range(7)]
            for cp in cps:
                cp.wait()
            for j in range(_REFA // _L):
                row = iota + (j * _L)
                sl = pl.ds(j * _L, _L)
                for k in range(7):
                    plsc.store_scatter(refb, [row, _i32(k)], rcols[k][sl])
            pltpu.sync_copy(refb, ref8.at[pl.ds(base, _REFA)])

    return pack


def _make_main(n):
    num_chunks = n // CHUNK

    @functools.partial(
        pl.kernel,
        mesh=_mesh(),
        compiler_params=pltpu.CompilerParams(
            needs_layout_passes=False,
            use_tc_tiling_on_sc=False,
        ),
        out_type=jax.ShapeDtypeStruct((2 * n,), jnp.float32),
        scratch_types=[
            pltpu.VMEM((8, 8), jnp.float32),
            pltpu.VMEM((8, 4), jnp.float32),
            [pltpu.VMEM((2 * CHUNK,), jnp.int32) for _ in range(2)],
            [pltpu.VMEM((CHUNK,), jnp.int32) for _ in range(2)],
            [pltpu.VMEM((CHUNK,), jnp.int32) for _ in range(2)],
            [pltpu.VMEM((CHUNK, 8), jnp.float32) for _ in range(2)],
            [pltpu.VMEM((CHUNK, 8), jnp.float32) for _ in range(2)],
            [pltpu.VMEM((2 * CHUNK,), jnp.float32) for _ in range(2)],
            [pltpu.VMEM((2 * CHUNK,), jnp.float32) for _ in range(2)],
            [pltpu.SemaphoreType.DMA for _ in range(2)],
            [pltpu.SemaphoreType.DMA for _ in range(2)],
            [pltpu.SemaphoreType.DMA for _ in range(2)],
        ],
    )
    def main(ref8, pt8, rel8, cam4, gm_flat, cidx, pidx, p2d_flat, out,
             rel_v, cam_v, gmbuf, pbuf, cbuf,
             ref_v, pt_v, p2buf, obuf, sem_in, sem_tab, sem_out):
        wid = lax.axis_index("s") * _NC + lax.axis_index("c")
        pltpu.sync_copy(rel8, rel_v)
        pltpu.sync_copy(cam4, cam_v)
        iota = lax.iota(jnp.int32, _L)
        nk = (num_chunks - wid + _NW - 1) // _NW

        def chunk_of(t):
            return wid + t * _NW

        def fire_lin(t, b):
            c = chunk_of(t)
            base2 = c * (2 * CHUNK)
            base = c * CHUNK
            pltpu.async_copy(gm_flat.at[pl.ds(base2, 2 * CHUNK)],
                             gmbuf[b], sem_in[b])
            pltpu.async_copy(pidx.at[pl.ds(base, CHUNK)], pbuf[b], sem_in[b])
            pltpu.async_copy(cidx.at[pl.ds(base, CHUNK)], cbuf[b], sem_in[b])
            pltpu.async_copy(p2d_flat.at[pl.ds(base2, 2 * CHUNK)],
                             p2buf[b], sem_in[b])

        def wait_lin(b):
            pltpu.make_async_copy(gm_flat.at[pl.ds(0, 2 * CHUNK)],
                                  gmbuf[b], sem_in[b]).wait()
            pltpu.make_async_copy(pidx.at[pl.ds(0, CHUNK)],
                                  pbuf[b], sem_in[b]).wait()
            pltpu.make_async_copy(cidx.at[pl.ds(0, CHUNK)],
                                  cbuf[b], sem_in[b]).wait()
            pltpu.make_async_copy(p2d_flat.at[pl.ds(0, 2 * CHUNK)],
                                  p2buf[b], sem_in[b]).wait()

        def fire_ind(b):
            for i in range(_SUB):
                pltpu.async_copy(
                    ref8.at[gmbuf[b].at[pl.ds(i * 2 * BLK, BLK)]],
                    ref_v[b].at[pl.ds(i * BLK, BLK)], sem_tab[b])
                pltpu.async_copy(
                    pt8.at[pbuf[b].at[pl.ds(i * BLK, BLK)]],
                    pt_v[b].at[pl.ds(i * BLK, BLK)], sem_tab[b])

        def wait_ind(b):
            for i in range(_SUB):
                pltpu.make_async_copy(
                    ref8.at[gmbuf[b].at[pl.ds(i * 2 * BLK, BLK)]],
                    ref_v[b].at[pl.ds(i * BLK, BLK)], sem_tab[b]).wait()
                pltpu.make_async_copy(
                    pt8.at[pbuf[b].at[pl.ds(i * BLK, BLK)]],
                    pt_v[b].at[pl.ds(i * BLK, BLK)], sem_tab[b]).wait()

        def fire_out(t, b):
            base2 = chunk_of(t) * (2 * CHUNK)
            pltpu.async_copy(obuf[b], out.at[pl.ds(base2, 2 * CHUNK)],
                             sem_out[b])

        def wait_out(b):
            pltpu.make_async_copy(obuf[b], out.at[pl.ds(0, 2 * CHUNK)],
                                  sem_out[b]).wait()

        def compute(b):
            @pl.loop(0, CHUNK // _L)
            def _grp(jj):
                blk = jj // 8
                jw = jj - blk * 8
                row = iota + (jj * _L)
                sl = pl.ds(jj * _L, _L)
                slx = pl.ds(blk * 2 * BLK + jw * _L, _L)
                sly = pl.ds(blk * 2 * BLK + BLK + jw * _L, _L)
                m = gmbuf[b][sly]
                cam = cbuf[b][sl]
                rv, pv = ref_v[b], pt_v[b]
                rt = tuple(plsc.load_gather(rv, [row, _i32(c)]) for c in range(3))
                rq = tuple(plsc.load_gather(rv, [row, _i32(c)]) for c in range(3, 7))
                lt = tuple(plsc.load_gather(rel_v, [m, _i32(c)]) for c in range(3))
                lq = tuple(plsc.load_gather(rel_v, [m, _i32(c)]) for c in range(3, 7))
                p = tuple(plsc.load_gather(pv, [row, _i32(c)]) for c in range(3))
                q_img = _qmul(lq, rq)
                rot = _qrot(lq, rt)
                t_img = (lt[0] + rot[0], lt[1] + rot[1], lt[2] + rot[2])
                pc = _qrot(q_img, p)
                pcx = pc[0] + t_img[0]
                pcy = pc[1] + t_img[1]
                inv_z = 1.0 / (pc[2] + t_img[2])
                fx = plsc.load_gather(cam_v, [cam, _i32(0)])
                fy = plsc.load_gather(cam_v, [cam, _i32(1)])
                ppx = plsc.load_gather(cam_v, [cam, _i32(2)])
                ppy = plsc.load_gather(cam_v, [cam, _i32(3)])
                obuf[b][slx] = fx * pcx * inv_z + ppx - p2buf[b][slx]
                obuf[b][sly] = fy * pcy * inv_z + ppy - p2buf[b][sly]

        fire_lin(0, 0)
        wait_lin(0)
        fire_ind(0)

        @pl.when(nk > 1)
        def _():
            fire_lin(1, 1)

        @pl.loop(0, (nk + 1) // 2 * 2, step=2)
        def _pair(t0):
            for b in range(2):
                t = t0 + b

                @pl.when(t < nk)
                def _():
                    nb = 1 - b

                    @pl.when(t + 1 < nk)
                    def _():
                        wait_lin(nb)
                        fire_ind(nb)

                    wait_ind(b)

                    @pl.when(t >= 2)
                    def _():
                        wait_out(b)

                    compute(b)
                    fire_out(t, b)

                    @pl.when(t + 2 < nk)
                    def _():
                        fire_lin(t + 2, b)

        @pl.when(jnp.logical_and(nk >= 2, (nk - 2) % 2 == 0))
        def _():
            wait_out(0)

        @pl.when(jnp.logical_and(nk >= 2, (nk - 2) % 2 == 1))
        def _():
            wait_out(1)

        @pl.when((nk - 1) % 2 == 0)
        def _():
            wait_out(0)

        @pl.when((nk - 1) % 2 == 1)
        def _():
            wait_out(1)

    return main


@jax.jit
def _run(points_2d, camera_indices, grouping_indices, point_indices,
         camera_pps, rel_poses, intrs, points_3d, ref_poses):
    n = points_2d.shape[0]
    nb = n // BLK
    npts = points_3d.shape[0]
    ngroups = ref_poses.shape[0]
    pack = _make_pack_tables(npts, ngroups)
    pt8, ref8 = pack(points_3d[:, 0], points_3d[:, 1], points_3d[:, 2],
                     *(ref_poses[:, i] for i in range(7)))
    rel8 = jnp.concatenate(
        [rel_poses, jnp.zeros((rel_poses.shape[0], 1), jnp.float32)], axis=1)
    cam4 = jnp.concatenate([intrs, camera_pps], axis=1)
    gm_flat = (grouping_indices.astype(jnp.int32)
               .reshape(nb, BLK, 2).transpose(0, 2, 1).reshape(-1))
    p2d_flat = points_2d.reshape(nb, BLK, 2).transpose(0, 2, 1).reshape(-1)
    main = _make_main(n)
    out_flat = main(ref8, pt8, rel8, cam4, gm_flat,
                    camera_indices.astype(jnp.int32),
                    point_indices.astype(jnp.int32),
                    p2d_flat)
    return out_flat.reshape(nb, 2, BLK).transpose(0, 2, 1).reshape(n, 2)


def kernel(points_2d, camera_indices, grouping_indices, point_indices,
           camera_pps, rel_poses, intrs, points_3d, ref_poses):
    return _run(points_2d, camera_indices, grouping_indices, point_indices,
                camera_pps, rel_poses, intrs, points_3d, ref_poses)
